# Initial kernel scaffold; baseline (speedup 1.0000x reference)
import jax
import jax.numpy as jnp
from jax.experimental import pallas as pl


def kernel(x, edge_index, batch, W0, b0, attr0, W1, b1, attr1, Wp1, bp1, Wp2, bp2):
    raise NotImplementedError("write your pallas kernel here")


# speedup vs baseline: 37.7133x; 37.7133x over previous
import functools

import jax
import jax.numpy as jnp
from jax import lax
from jax.experimental import pallas as pl
from jax.experimental.pallas import tpu as pltpu
from jax.experimental.pallas import tpu_sc as plsc

N = 10000
E = 320000
D = 128
H = 2
C = 64
HC = H * C
EPS = 0.01
NEG_SLOPE = 0.2

NS = 16
EPT = E // NS
K = 80
NCHUNK = EPT // K
NP = 10240
ROWS_PT = NP // NS
AW = 80


def _sc_gat_edges(table_hbm, src_hbm, dst_hbm, attr_hbm, zeros_hbm, out_hbm,
                  src_v, dsta_v, dst_v, xj_v, xi_v, msg_v,
                  attr_v, accum, sem_j, sem_i):
    cid = lax.axis_index("c")
    sid = lax.axis_index("s")

    pltpu.sync_copy(zeros_hbm.at[pl.ds(sid * ROWS_PT, ROWS_PT)],
                    accum.at[pl.ds(sid * ROWS_PT, ROWS_PT)])
    pltpu.sync_copy(attr_hbm.at[cid], attr_v)
    plsc.subcore_barrier()

    row_off = cid * N

    def chunk_body(ci, carry):
        base = sid * EPT + ci * K
        pltpu.sync_copy(src_hbm.at[pl.ds(base, K)], src_v)
        pltpu.sync_copy(dst_hbm.at[pl.ds(base, K)], dst_v)
        for i in range(K // 16):
            sl = pl.ds(i * 16, 16)
            src_v[sl] = src_v[sl] + row_off
            dsta_v[sl] = dst_v[sl] + row_off
        cj = pltpu.async_copy(table_hbm.at[src_v], xj_v, sem_j)
        ci_ = pltpu.async_copy(table_hbm.at[dsta_v], xi_v, sem_i)
        cj.wait()
        ci_.wait()

        attr_q = [attr_v[pl.ds(q * 16, 16)] for q in range(C // 16)]
        for e in range(K):
            s = None
            for q in range(C // 16):
                sl = pl.ds(q * 16, 16)
                m = xj_v[e, sl] + (1.0 + EPS) * xi_v[e, sl]
                lr = jnp.maximum(m, NEG_SLOPE * m)
                t = lr * attr_q[q]
                s = t if s is None else s + t
            alpha = jnp.sum(s)
            pb = jnp.exp(jnp.full((16,), alpha, dtype=jnp.float32))
            for q in range(C // 16):
                sl = pl.ds(q * 16, 16)
                msg_v[e, sl] = xj_v[e, sl] * pb
            msg_v[e, pl.ds(C, 16)] = pb
        pltpu.sync_copy(msg_v, accum.at[dst_v], add=True)
        return carry

    lax.fori_loop(0, NCHUNK, chunk_body, 0)
    plsc.subcore_barrier()
    pltpu.sync_copy(accum.at[pl.ds(sid * ROWS_PT, ROWS_PT)],
                    out_hbm.at[cid, pl.ds(sid * ROWS_PT, ROWS_PT)])


@functools.cache
def _sc_gat_kernel():
    return pl.kernel(
        _sc_gat_edges,
        out_type=jax.ShapeDtypeStruct((H, NP, AW), jnp.float32),
        mesh=plsc.VectorSubcoreMesh(core_axis_name="c", subcore_axis_name="s",
                                    num_cores=H, num_subcores=NS),
        compiler_params=pltpu.CompilerParams(use_tc_tiling_on_sc=False,
                                             needs_layout_passes=False),
        scratch_types=[
            pltpu.VMEM((K,), jnp.int32),
            pltpu.VMEM((K,), jnp.int32),
            pltpu.VMEM((K,), jnp.int32),
            pltpu.VMEM((K, C), jnp.float32),
            pltpu.VMEM((K, C), jnp.float32),
            pltpu.VMEM((K, AW), jnp.float32),
            pltpu.VMEM((C,), jnp.float32),
            pltpu.VMEM_SHARED((NP, AW), jnp.float32),
            pltpu.SemaphoreType.DMA,
            pltpu.SemaphoreType.DMA,
        ],
    )


def _tc_in_body(x_ref, w_ref, b_ref, out_ref):
    y = jnp.dot(x_ref[...], w_ref[...].T, preferred_element_type=jnp.float32)
    y = y + b_ref[...]
    out_ref[pl.ds(0, N), :] = y[:, 0:C]
    out_ref[pl.ds(N, N), :] = y[:, C:HC]


def _tc_mid_body(acc_ref, w_ref, b_ref, out_ref):
    num0 = acc_ref[0, 0:N, 0:C]
    den0 = acc_ref[0, 0:N, C:C + 1]
    num1 = acc_ref[1, 0:N, 0:C]
    den1 = acc_ref[1, 0:N, C:C + 1]
    h0 = jnp.maximum(num0 / (den0 + 1e-16), 0.0)
    h1 = jnp.maximum(num1 / (den1 + 1e-16), 0.0)
    h = jnp.concatenate([h0, h1], axis=1)
    y = jnp.dot(h, w_ref[...].T, preferred_element_type=jnp.float32)
    y = y + b_ref[...]
    out_ref[pl.ds(0, N), :] = y[:, 0:C]
    out_ref[pl.ds(N, N), :] = y[:, C:HC]


def _tc_head_body(acc_ref, w1_ref, b1_ref, w2_ref, b2_ref, out_ref):
    num0 = acc_ref[0, 0:N, 0:C]
    den0 = acc_ref[0, 0:N, C:C + 1]
    num1 = acc_ref[1, 0:N, 0:C]
    den1 = acc_ref[1, 0:N, C:C + 1]
    h0 = jnp.maximum(num0 / (den0 + 1e-16), 0.0)
    h1 = jnp.maximum(num1 / (den1 + 1e-16), 0.0)
    h = jnp.concatenate([h0, h1], axis=1)
    y = jnp.dot(h, w1_ref[...].T, preferred_element_type=jnp.float32)
    y = y + b1_ref[...]
    y = jnp.dot(y, w2_ref[...].T, preferred_element_type=jnp.float32)
    y = y + b2_ref[...]
    m = jnp.max(y, axis=1, keepdims=True)
    z = y - m
    out_ref[...] = z - jnp.log(jnp.sum(jnp.exp(z), axis=1, keepdims=True))


_tc_in = pl.pallas_call(
    _tc_in_body,
    out_shape=jax.ShapeDtypeStruct((H * N, C), jnp.float32),
)

_tc_mid = pl.pallas_call(
    _tc_mid_body,
    out_shape=jax.ShapeDtypeStruct((H * N, C), jnp.float32),
)

_tc_head = pl.pallas_call(
    _tc_head_body,
    out_shape=jax.ShapeDtypeStruct((N, C), jnp.float32),
)


def kernel(x, edge_index, batch, W0, b0, attr0, W1, b1, attr1, Wp1, bp1, Wp2, bp2):
    src = edge_index[0]
    dst = edge_index[1]
    zeros = jnp.zeros((NP, AW), dtype=jnp.float32)
    attr0f = attr0.reshape(H, C)
    attr1f = attr1.reshape(H, C)

    sc_gat = _sc_gat_kernel()
    table0 = _tc_in(x, W0, b0.reshape(1, HC))
    acc0 = sc_gat(table0, src, dst, attr0f, zeros)
    table1 = _tc_mid(acc0, W1, b1.reshape(1, HC))
    acc1 = sc_gat(table1, src, dst, attr1f, zeros)
    return _tc_head(acc1, Wp1, bp1.reshape(1, C), Wp2, bp2.reshape(1, C))
